# baseline (speedup 1.0000x reference)
import jax
import jax.numpy as jnp
from jax.experimental import pallas as pl

SEQ = 9
DM = 1024
EPS = 1e-7


def _add_enc_kernel(x_ref, row_ref, o_ref):
    row = row_ref[...]
    norm = jnp.sqrt(jnp.sum(row * row))
    scale = jnp.where(norm > 1.0, 1.0 / (norm + EPS), 1.0)
    o_ref[...] = x_ref[...] + row * scale


def kernel(x, table):
    b, s, n, d = x.shape
    xr = x.reshape(b * s, n, d)
    tr = table.reshape(SEQ, 1, d)
    rows = 512
    out = pl.pallas_call(
        _add_enc_kernel,
        grid=(b * s, n // rows),
        in_specs=[
            pl.BlockSpec((1, rows, d), lambda i, j: (i, j, 0)),
            pl.BlockSpec((1, 1, d), lambda i, j: (i % SEQ, 0, 0)),
        ],
        out_specs=pl.BlockSpec((1, rows, d), lambda i, j: (i, j, 0)),
        out_shape=jax.ShapeDtypeStruct((b * s, n, d), x.dtype),
    )(xr, tr)
    return out.reshape(b, s, n, d)


# speedup vs baseline: 1.1567x; 1.1567x over previous
import jax
import jax.numpy as jnp
from jax.experimental import pallas as pl
from jax.experimental.pallas import tpu as pltpu

SEQ = 9
DM = 1024
EPS = 1e-7


def _add_enc_kernel(x_ref, row_ref, o_ref):
    row = row_ref[...]
    norm = jnp.sqrt(jnp.sum(row * row))
    scale = jnp.where(norm > 1.0, 1.0 / (norm + EPS), 1.0)
    o_ref[...] = x_ref[...] + row * scale


def kernel(x, table):
    b, s, n, d = x.shape
    xr = x.reshape(b * s, n, d)
    tr = table.reshape(SEQ, 1, d)
    out = pl.pallas_call(
        _add_enc_kernel,
        grid=(b * s,),
        in_specs=[
            pl.BlockSpec((1, n, d), lambda i: (i, 0, 0)),
            pl.BlockSpec((1, 1, d), lambda i: (i % SEQ, 0, 0)),
        ],
        out_specs=pl.BlockSpec((1, n, d), lambda i: (i, 0, 0)),
        out_shape=jax.ShapeDtypeStruct((b * s, n, d), x.dtype),
        compiler_params=pltpu.CompilerParams(
            dimension_semantics=("arbitrary",),
            vmem_limit_bytes=60 * 1024 * 1024,
        ),
    )(xr, tr)
    return out.reshape(b, s, n, d)
